# baseline (speedup 1.0000x reference)
import functools

import jax
import jax.numpy as jnp
from jax import lax
from jax.experimental import pallas as pl
from jax.experimental.pallas import tpu as pltpu
from jax.experimental.pallas import tpu_sc as plsc

B, C, V = 4096, 200, 128
NT, NE = 128, 64
NC, NS, LANES = 2, 16, 16
NW = NC * NS
ROWS = B * C
RPW = ROWS // NW
CHUNK = 128
NCHUNK = RPW // CHUNK
NBUF = 4
GROUPS = NCHUNK // NBUF
PERIOD = 25


def _prep_body(tm_ref, mm_ref, tab_ref, fus_ref, pos_ref, cpat_ref):
    onehot = (tm_ref[...][:, None] == lax.broadcasted_iota(jnp.int32, (NT, NE), 1))
    fused = lax.dot(onehot.astype(jnp.float32), tab_ref[...],
                    precision=lax.Precision.HIGHEST,
                    preferred_element_type=jnp.float32)
    fus_ref[...] = fused * mm_ref[...][:, None]
    pos_ref[...] = lax.broadcasted_iota(jnp.int32, (C, 1), 0).astype(jnp.float32)
    flat = (lax.broadcasted_iota(jnp.int32, (PERIOD, CHUNK), 0) * CHUNK
            + lax.broadcasted_iota(jnp.int32, (PERIOD, CHUNK), 1))
    cpat_ref[...] = lax.rem(flat, C)


def _build_prep(token_map, mult_map, table):
    return pl.pallas_call(
        _prep_body,
        out_shape=(
            jax.ShapeDtypeStruct((NT, V), jnp.float32),
            jax.ShapeDtypeStruct((C, 1), jnp.float32),
            jax.ShapeDtypeStruct((PERIOD, CHUNK), jnp.int32),
        ),
    )(token_map, mult_map, table)


def _sc_body(fus_hbm, pos_hbm, cpat_hbm, ch_hbm, inp_hbm, out_hbm,
             g1s0, g1s1, g1s2, g1s3, g2s0, g2s1, g2s2, g2s3,
             ss0, ss1, ss2, ss3):
    pl.run_scoped(
        functools.partial(
            _sc_inner, fus_hbm, pos_hbm, cpat_hbm, ch_hbm, inp_hbm, out_hbm,
            (g1s0, g1s1, g1s2, g1s3), (g2s0, g2s1, g2s2, g2s3),
            (ss0, ss1, ss2, ss3)),
        pltpu.VMEM((NCHUNK, CHUNK), jnp.int32),
        pltpu.VMEM((NBUF * CHUNK, V), jnp.float32),
        pltpu.VMEM((PERIOD, CHUNK), jnp.int32),
        pltpu.VMEM((LANES,), jnp.int32),
    )


def _sc_inner(fus_hbm, pos_hbm, cpat_hbm, ch_hbm, inp_hbm, out_hbm,
              g1sems, g2sems, ssems, idx2, bigbuf, cpat_v, ch_v):
    wid = lax.axis_index("s") * NC + lax.axis_index("c")

    pltpu.sync_copy(inp_hbm.at[wid], idx2)
    pltpu.sync_copy(cpat_hbm, cpat_v)
    pltpu.sync_copy(ch_hbm, ch_v)
    chvec = ch_v[...]
    chs = tuple(chvec[l] for l in range(LANES))

    def fire_gather(g, b):
        pltpu.async_copy(fus_hbm.at[idx2.at[g]],
                         bigbuf.at[pl.ds(b * CHUNK, CHUNK)], g1sems[b])

    def wait_gather(b):
        pltpu.make_async_copy(fus_hbm.at[idx2.at[0]],
                              bigbuf.at[pl.ds(0, CHUNK)], g1sems[b]).wait()

    def fire_splices(g, b):
        gm = lax.rem(g, PERIOD)
        for l in range(LANES):
            ch = chs[l]

            @pl.when(ch < V)
            def _():
                pltpu.async_copy(
                    pos_hbm.at[cpat_v.at[gm]],
                    bigbuf.at[pl.ds(b * CHUNK, CHUNK), pl.ds(ch, 1)],
                    g2sems[b])

    def wait_splices(b):
        for l in range(LANES):
            ch = chs[l]

            @pl.when(ch < V)
            def _():
                pltpu.make_async_copy(
                    pos_hbm.at[cpat_v.at[0]],
                    bigbuf.at[pl.ds(0, CHUNK), pl.ds(0, 1)],
                    g2sems[b]).wait()

    def fire_scatter(g, b):
        pltpu.async_copy(bigbuf.at[pl.ds(b * CHUNK, CHUNK)],
                         out_hbm.at[wid, g], ssems[b])

    def wait_scatter(b):
        pltpu.make_async_copy(bigbuf.at[pl.ds(0, CHUNK)],
                              out_hbm.at[0, 0], ssems[b]).wait()

    for b in range(NBUF):
        fire_gather(b, b)

    def group(p, carry):
        for b in range(NBUF):
            g = p * NBUF + b
            wait_gather(b)
            fire_scatter(p * NBUF + b, b)
        for b in range(NBUF):
            g = p * NBUF + b
            wait_scatter(b)
            fire_gather(g + NBUF, b)
        return carry

    lax.fori_loop(0, GROUPS - 1, group, 0)

    last = (GROUPS - 1) * NBUF
    for b in range(NBUF):
        wait_gather(b)
        fire_scatter(last + b, b)
    for b in range(NBUF):
        wait_scatter(b)


@jax.jit
def kernel(input_BC, token_map, mult_map, table, channel_mask):
    fused, pos_c, cpat = _build_prep(token_map, mult_map, table)
    ch_ids = jnp.nonzero(channel_mask, size=LANES, fill_value=V)[0].astype(jnp.int32)
    inp3d = input_BC.reshape(NW, NCHUNK, CHUNK)

    sc = pl.kernel(
        _sc_body,
        out_type=jax.ShapeDtypeStruct((NW, NCHUNK, CHUNK, V), jnp.float32),
        mesh=plsc.VectorSubcoreMesh(core_axis_name="c", subcore_axis_name="s"),
        scratch_types=[pltpu.SemaphoreType.DMA] * 12,
    )
    out4 = sc(fused, pos_c, cpat, ch_ids, inp3d)
    return out4.reshape(B, C, V)


# speedup vs baseline: 2.9584x; 2.9584x over previous
import functools

import jax
import jax.numpy as jnp
from jax import lax
from jax.experimental import pallas as pl
from jax.experimental.pallas import tpu as pltpu
from jax.experimental.pallas import tpu_sc as plsc

B, C, V = 4096, 200, 128
NT, NE = 128, 64
NC, NS, LANES = 2, 16, 16
NW = NC * NS
ROWS = B * C
RPW = ROWS // NW
CHUNK = 128
NCHUNK = RPW // CHUNK
NBUF = 4
GROUPS = NCHUNK // NBUF

CB = 8


def _build_body(tm_ref, mm_ref, tab_ref, cm_ref, out_ref):
    i = pl.program_id(0)
    tm = tm_ref[...]
    mm = mm_ref[...]
    onehot = (tm[:, None] == lax.broadcasted_iota(jnp.int32, (NT, NE), 1))
    fused = lax.dot(onehot.astype(jnp.float32), tab_ref[...],
                    precision=lax.Precision.HIGHEST,
                    preferred_element_type=jnp.float32)
    fused = fused * mm[:, None]
    cpos = (lax.broadcasted_iota(jnp.int32, (NT, CB, V), 1) + i * CB).astype(jnp.float32)
    masked = cm_ref[...][None, None, :] != 0
    out_ref[...] = jnp.where(masked, cpos, fused[:, None, :])


def _build_expanded(token_map, mult_map, table, channel_mask):
    return pl.pallas_call(
        _build_body,
        grid=(C // CB,),
        in_specs=[
            pl.BlockSpec((NT,), lambda i: (0,)),
            pl.BlockSpec((NT,), lambda i: (0,)),
            pl.BlockSpec((NE, V), lambda i: (0, 0)),
            pl.BlockSpec((V,), lambda i: (0,)),
        ],
        out_specs=pl.BlockSpec((NT, CB, V), lambda i: (0, i, 0)),
        out_shape=jax.ShapeDtypeStruct((NT, C, V), jnp.float32),
    )(token_map, mult_map, table, channel_mask.astype(jnp.int32))


def _sc_body(exp_hbm, inp_hbm, out_hbm, idx2,
             b0, b1, b2, b3, g0, g1, g2, g3, s0, s1, s2, s3):
    bufs = (b0, b1, b2, b3)
    gsems = (g0, g1, g2, g3)
    ssems = (s0, s1, s2, s3)
    wid = lax.axis_index("s") * NC + lax.axis_index("c")

    pltpu.sync_copy(inp_hbm.at[wid], idx2)

    iota16 = lax.broadcasted_iota(jnp.int32, (LANES,), 0)

    def prep(g):
        for j in range(CHUNK // LANES):
            raw = idx2[g, pl.ds(j * LANES, LANES)]
            pos = lax.rem(g * CHUNK + j * LANES + iota16, C)
            idx2[g, pl.ds(j * LANES, LANES)] = raw * C + pos

    def fire_gather(g, b):
        prep(g)
        pltpu.async_copy(exp_hbm.at[idx2.at[g]], bufs[b], gsems[b])

    def wait_gather(g, b):
        pltpu.make_async_copy(exp_hbm.at[idx2.at[g]], bufs[b], gsems[b]).wait()

    def fire_scatter(g, b):
        pltpu.async_copy(bufs[b], out_hbm.at[wid, g], ssems[b])

    def wait_scatter(g, b):
        pltpu.make_async_copy(bufs[b], out_hbm.at[wid, g], ssems[b]).wait()

    for b in range(NBUF):
        fire_gather(b, b)

    def group(p, carry):
        for b in range(NBUF):
            g = p * NBUF + b
            wait_gather(g, b)
            fire_scatter(g, b)
        for b in range(NBUF):
            g = p * NBUF + b
            wait_scatter(g, b)
            fire_gather(g + NBUF, b)
        return carry

    lax.fori_loop(0, GROUPS - 1, group, 0)

    last = (GROUPS - 1) * NBUF
    for b in range(NBUF):
        wait_gather(last + b, b)
        fire_scatter(last + b, b)
    for b in range(NBUF):
        wait_scatter(last + b, b)


@functools.partial(jax.jit, static_argnames=())
def kernel(input_BC, token_map, mult_map, table, channel_mask):
    expanded = _build_expanded(token_map, mult_map, table, channel_mask)
    exp2d = expanded.reshape(NT * C, V)
    inp3d = input_BC.reshape(NW, NCHUNK, CHUNK)

    gather = pl.kernel(
        _sc_body,
        out_type=jax.ShapeDtypeStruct((NW, NCHUNK, CHUNK, V), jnp.float32),
        mesh=plsc.VectorSubcoreMesh(core_axis_name="c", subcore_axis_name="s"),
        scratch_types=[
            pltpu.VMEM((NCHUNK, CHUNK), jnp.int32),
            pltpu.VMEM((CHUNK, V), jnp.float32),
            pltpu.VMEM((CHUNK, V), jnp.float32),
            pltpu.VMEM((CHUNK, V), jnp.float32),
            pltpu.VMEM((CHUNK, V), jnp.float32),
            pltpu.SemaphoreType.DMA,
            pltpu.SemaphoreType.DMA,
            pltpu.SemaphoreType.DMA,
            pltpu.SemaphoreType.DMA,
            pltpu.SemaphoreType.DMA,
            pltpu.SemaphoreType.DMA,
            pltpu.SemaphoreType.DMA,
            pltpu.SemaphoreType.DMA,
        ],
    )
    out4 = gather(exp2d, inp3d)
    return out4.reshape(B, C, V)
